# baseline (device time: 38903 ns/iter reference)
import jax
import jax.numpy as jnp
from jax import lax
from jax.experimental import pallas as pl
from jax.experimental.pallas import tpu as pltpu

SUB = 8


def kernel(x, W):
    t, d = x.shape
    _, v_half = W.shape
    qcols = v_half // 4
    hcols = qcols // 2
    w = qcols // SUB

    def body(x_ref, w_ref, out_ref, logits, zsend, zrecv, xrecv, yrecv,
             drecv, ezq, exq, eyq, stats_send, stats_recv, zs_sems, zr_sems,
             x1s_sems, x1r_sems, y1s_sems, y1r_sems, p2_sems, stats_sems,
             out_sems):
        my_x = lax.axis_index("x")
        my_y = lax.axis_index("y")
        my_z = lax.axis_index("z")
        peer_z = 1 - my_z
        q_me = 2 * my_x + my_y
        q_x = 2 * (1 - my_x) + my_y
        q_y = 2 * my_x + (1 - my_y)
        q_d = 2 * (1 - my_x) + (1 - my_y)

        z_peer = (my_x, my_y, peer_z)
        x_nbr = (1 - my_x, my_y, my_z)
        y_nbr = (my_x, 1 - my_y, my_z)

        barrier_sem = pltpu.get_barrier_semaphore()
        for dev in [z_peer, x_nbr, y_nbr]:
            pl.semaphore_signal(
                barrier_sem, inc=1,
                device_id=dev, device_id_type=pl.DeviceIdType.MESH,
            )

        def sub(buf, s):
            return buf.at[:, s * w:(s + 1) * w]

        def copy(src, dst, send_sem, recv_sem, dev):
            return pltpu.make_async_remote_copy(
                src_ref=src, dst_ref=dst,
                send_sem=send_sem, recv_sem=recv_sem,
                device_id=dev, device_id_type=pl.DeviceIdType.MESH,
            )

        z_rdmas = [
            copy(sub(zsend, s), sub(zrecv, s), zs_sems.at[s], zr_sems.at[s],
                 z_peer)
            for s in range(SUB)
        ]
        x1_rdmas = [
            copy(sub(zrecv, s), sub(xrecv, s), x1s_sems.at[s],
                 x1r_sems.at[s], x_nbr)
            for s in range(SUB)
        ]
        y1_rdmas = [
            copy(sub(zrecv, s), sub(yrecv, s), y1s_sems.at[s],
                 y1r_sems.at[s], y_nbr)
            for s in range(SUB)
        ]
        x2_rdma = copy(yrecv.at[:, 0:hcols], drecv.at[:, 0:hcols],
                       p2_sems.at[0], p2_sems.at[1], x_nbr)
        y2_rdma = copy(xrecv.at[:, hcols:qcols], drecv.at[:, hcols:qcols],
                       p2_sems.at[2], p2_sems.at[3], y_nbr)
        stats_rdma = pltpu.make_async_remote_copy(
            src_ref=stats_send, dst_ref=stats_recv,
            send_sem=stats_sems.at[0], recv_sem=stats_sems.at[1],
            device_id=z_peer, device_id_type=pl.DeviceIdType.MESH,
        )

        logits[:, :] = jnp.dot(x_ref[:, :], w_ref[:, :],
                               preferred_element_type=jnp.float32)
        zsend[:, :] = logits[:, pl.ds(q_me * qcols, qcols)]
        m_l = jnp.max(logits[:, :], axis=-1, keepdims=True)
        s_l = jnp.sum(jnp.exp(logits[:, :] - m_l), axis=-1, keepdims=True)
        stats_send[:, 0:1] = m_l
        stats_send[:, 1:2] = s_l
        pl.semaphore_wait(barrier_sem, 3)
        z_rdmas[0].start()
        z_rdmas[1].start()
        stats_rdma.start()
        for s in range(2, SUB):
            z_rdmas[s].start()

        for s in range(SUB):
            z_rdmas[s].wait_recv()
            x1_rdmas[s].start()
            y1_rdmas[s].start()

        for s in range(SUB // 2):
            y1_rdmas[s].wait_recv()
        x2_rdma.start()

        stats_rdma.wait_recv()
        m_p = stats_recv[:, 0:1]
        s_p = stats_recv[:, 1:2]
        m = jnp.maximum(m_l, m_p)
        inv = 1.0 / (s_l * jnp.exp(m_l - m) + s_p * jnp.exp(m_p - m))
        logits[:, :] = jnp.exp(logits[:, :] - m) * inv
        cp_local = pltpu.make_async_copy(
            logits, out_ref.at[:, pl.ds(my_z * v_half, v_half)],
            out_sems.at[0])
        cp_local.start()
        peer_base = peer_z * v_half
        ezq[:, :] = jnp.exp(zrecv[:, :] - m) * inv
        cp_zq = pltpu.make_async_copy(
            ezq, out_ref.at[:, pl.ds(peer_base + q_me * qcols, qcols)],
            out_sems.at[1])
        cp_zq.start()

        for s in range(SUB // 2, SUB):
            x1_rdmas[s].wait_recv()
        y2_rdma.start()

        for s in range(SUB // 2):
            x1_rdmas[s].wait_recv()
        exq[:, :] = jnp.exp(xrecv[:, :] - m) * inv
        cp_xq = pltpu.make_async_copy(
            exq, out_ref.at[:, pl.ds(peer_base + q_x * qcols, qcols)],
            out_sems.at[2])
        cp_xq.start()
        for s in range(SUB // 2, SUB):
            y1_rdmas[s].wait_recv()
        eyq[:, :] = jnp.exp(yrecv[:, :] - m) * inv
        cp_yq = pltpu.make_async_copy(
            eyq, out_ref.at[:, pl.ds(peer_base + q_y * qcols, qcols)],
            out_sems.at[3])
        cp_yq.start()

        x2_rdma.wait_recv()
        y2_rdma.wait_recv()
        drecv[:, :] = jnp.exp(drecv[:, :] - m) * inv
        cp_dq = pltpu.make_async_copy(
            drecv, out_ref.at[:, pl.ds(peer_base + q_d * qcols, qcols)],
            out_sems.at[4])
        cp_dq.start()

        for rdma in z_rdmas + x1_rdmas + y1_rdmas:
            rdma.wait_send()
        x2_rdma.wait_send()
        y2_rdma.wait_send()
        stats_rdma.wait_send()
        for cp in [cp_local, cp_zq, cp_xq, cp_yq, cp_dq]:
            cp.wait()

    return pl.pallas_call(
        body,
        out_shape=jax.ShapeDtypeStruct((t, 2 * v_half), jnp.float32),
        in_specs=[
            pl.BlockSpec(memory_space=pltpu.VMEM),
            pl.BlockSpec(memory_space=pltpu.VMEM),
        ],
        out_specs=pl.BlockSpec(memory_space=pl.ANY),
        scratch_shapes=[
            pltpu.VMEM((t, v_half), jnp.float32),
            pltpu.VMEM((t, qcols), jnp.float32),
            pltpu.VMEM((t, qcols), jnp.float32),
            pltpu.VMEM((t, qcols), jnp.float32),
            pltpu.VMEM((t, qcols), jnp.float32),
            pltpu.VMEM((t, qcols), jnp.float32),
            pltpu.VMEM((t, qcols), jnp.float32),
            pltpu.VMEM((t, qcols), jnp.float32),
            pltpu.VMEM((t, qcols), jnp.float32),
            pltpu.VMEM((t, 8), jnp.float32),
            pltpu.VMEM((t, 8), jnp.float32),
            pltpu.SemaphoreType.DMA((SUB,)),
            pltpu.SemaphoreType.DMA((SUB,)),
            pltpu.SemaphoreType.DMA((SUB,)),
            pltpu.SemaphoreType.DMA((SUB,)),
            pltpu.SemaphoreType.DMA((SUB,)),
            pltpu.SemaphoreType.DMA((SUB,)),
            pltpu.SemaphoreType.DMA((4,)),
            pltpu.SemaphoreType.DMA((2,)),
            pltpu.SemaphoreType.DMA((5,)),
        ],
        compiler_params=pltpu.CompilerParams(collective_id=0),
    )(x, W)


# device time: 35814 ns/iter; 1.0863x vs baseline; 1.0863x over previous
import jax
import jax.numpy as jnp
from jax import lax
from jax.experimental import pallas as pl
from jax.experimental.pallas import tpu as pltpu

R_COLS = 512
SUBU = 7
DX_COLS = 512


def kernel(x, W):
    t, d = x.shape
    _, v_half = W.shape
    ucols = (v_half - R_COLS) // 4
    w = ucols // SUBU
    dy_cols = ucols - DX_COLS

    def body(x_ref, w_ref, out_ref, logits, usend, rsend, urecv, rrecv,
             xrecv, yrecv, drecv, stats_send, stats_recv, zs_sems, zr_sems,
             x1s_sems, x1r_sems, y1s_sems, y1r_sems, zR_sems, p2_sems,
             stats_sems):
        my_x = lax.axis_index("x")
        my_y = lax.axis_index("y")
        my_z = lax.axis_index("z")
        peer_z = 1 - my_z
        q_me = 2 * my_x + my_y
        q_x = 2 * (1 - my_x) + my_y
        q_y = 2 * my_x + (1 - my_y)
        q_d = 2 * (1 - my_x) + (1 - my_y)

        z_peer = (my_x, my_y, peer_z)
        x_nbr = (1 - my_x, my_y, my_z)
        y_nbr = (my_x, 1 - my_y, my_z)

        barrier_sem = pltpu.get_barrier_semaphore()
        for dev in [z_peer, x_nbr, y_nbr]:
            pl.semaphore_signal(
                barrier_sem, inc=1,
                device_id=dev, device_id_type=pl.DeviceIdType.MESH,
            )

        def sub(buf, s):
            return buf.at[:, s * w:(s + 1) * w]

        def copy(src, dst, send_sem, recv_sem, dev):
            return pltpu.make_async_remote_copy(
                src_ref=src, dst_ref=dst,
                send_sem=send_sem, recv_sem=recv_sem,
                device_id=dev, device_id_type=pl.DeviceIdType.MESH,
            )

        z_rdmas = [
            copy(sub(usend, s), sub(urecv, s), zs_sems.at[s], zr_sems.at[s],
                 z_peer)
            for s in range(SUBU)
        ]
        zR_rdma = copy(rsend, rrecv, zR_sems.at[0], zR_sems.at[1], z_peer)
        x1_rdmas = [
            copy(sub(urecv, s), sub(xrecv, s), x1s_sems.at[s],
                 x1r_sems.at[s], x_nbr)
            for s in range(SUBU)
        ]
        y1_rdmas = [
            copy(sub(urecv, s), sub(yrecv, s), y1s_sems.at[s],
                 y1r_sems.at[s], y_nbr)
            for s in range(SUBU)
        ]
        x2_rdma = copy(yrecv.at[:, 0:DX_COLS], drecv.at[:, 0:DX_COLS],
                       p2_sems.at[0], p2_sems.at[1], x_nbr)
        y2_rdma = copy(xrecv.at[:, DX_COLS:ucols], drecv.at[:, DX_COLS:ucols],
                       p2_sems.at[2], p2_sems.at[3], y_nbr)
        stats_rdma = pltpu.make_async_remote_copy(
            src_ref=stats_send, dst_ref=stats_recv,
            send_sem=stats_sems.at[0], recv_sem=stats_sems.at[1],
            device_id=z_peer, device_id_type=pl.DeviceIdType.MESH,
        )

        logits[:, :] = jnp.dot(x_ref[:, :], w_ref[:, :],
                               preferred_element_type=jnp.float32)
        usend[:, :] = logits[:, pl.ds(R_COLS + q_me * ucols, ucols)]
        rsend[:, :] = logits[:, 0:R_COLS]
        m_l = jnp.max(logits[:, :], axis=-1, keepdims=True)
        s_l = jnp.sum(jnp.exp(logits[:, :] - m_l), axis=-1, keepdims=True)
        stats_send[:, 0:1] = m_l
        stats_send[:, 1:2] = s_l
        pl.semaphore_wait(barrier_sem, 3)
        z_rdmas[0].start()
        z_rdmas[1].start()
        stats_rdma.start()
        for s in range(2, SUBU):
            z_rdmas[s].start()
        zR_rdma.start()

        for s in range(SUBU):
            z_rdmas[s].wait_recv()
            x1_rdmas[s].start()
            y1_rdmas[s].start()

        for s in range(DX_COLS // w):
            y1_rdmas[s].wait_recv()
        x2_rdma.start()

        stats_rdma.wait_recv()
        m_p = stats_recv[:, 0:1]
        s_p = stats_recv[:, 1:2]
        m = jnp.maximum(m_l, m_p)
        inv = 1.0 / (s_l * jnp.exp(m_l - m) + s_p * jnp.exp(m_p - m))
        out_ref[:, pl.ds(my_z * v_half, v_half)] = (
            jnp.exp(logits[:, :] - m) * inv)
        peer_base = peer_z * v_half
        out_ref[:, pl.ds(peer_base + R_COLS + q_me * ucols, ucols)] = (
            jnp.exp(urecv[:, :] - m) * inv)

        for s in range(DX_COLS // w, SUBU):
            x1_rdmas[s].wait_recv()
        y2_rdma.start()

        for s in range(DX_COLS // w):
            x1_rdmas[s].wait_recv()
        out_ref[:, pl.ds(peer_base + R_COLS + q_x * ucols, ucols)] = (
            jnp.exp(xrecv[:, :] - m) * inv)
        for s in range(DX_COLS // w, SUBU):
            y1_rdmas[s].wait_recv()
        out_ref[:, pl.ds(peer_base + R_COLS + q_y * ucols, ucols)] = (
            jnp.exp(yrecv[:, :] - m) * inv)

        zR_rdma.wait_recv()
        out_ref[:, pl.ds(peer_base, R_COLS)] = (
            jnp.exp(rrecv[:, :] - m) * inv)

        x2_rdma.wait_recv()
        y2_rdma.wait_recv()
        out_ref[:, pl.ds(peer_base + R_COLS + q_d * ucols, ucols)] = (
            jnp.exp(drecv[:, :] - m) * inv)

        for rdma in z_rdmas + x1_rdmas + y1_rdmas:
            rdma.wait_send()
        zR_rdma.wait_send()
        x2_rdma.wait_send()
        y2_rdma.wait_send()
        stats_rdma.wait_send()

    return pl.pallas_call(
        body,
        out_shape=jax.ShapeDtypeStruct((t, 2 * v_half), jnp.float32),
        in_specs=[
            pl.BlockSpec(memory_space=pltpu.VMEM),
            pl.BlockSpec(memory_space=pltpu.VMEM),
        ],
        out_specs=pl.BlockSpec(memory_space=pltpu.VMEM),
        scratch_shapes=[
            pltpu.VMEM((t, v_half), jnp.float32),
            pltpu.VMEM((t, ucols), jnp.float32),
            pltpu.VMEM((t, R_COLS), jnp.float32),
            pltpu.VMEM((t, ucols), jnp.float32),
            pltpu.VMEM((t, R_COLS), jnp.float32),
            pltpu.VMEM((t, ucols), jnp.float32),
            pltpu.VMEM((t, ucols), jnp.float32),
            pltpu.VMEM((t, ucols), jnp.float32),
            pltpu.VMEM((t, 8), jnp.float32),
            pltpu.VMEM((t, 8), jnp.float32),
            pltpu.SemaphoreType.DMA((SUBU,)),
            pltpu.SemaphoreType.DMA((SUBU,)),
            pltpu.SemaphoreType.DMA((SUBU,)),
            pltpu.SemaphoreType.DMA((SUBU,)),
            pltpu.SemaphoreType.DMA((SUBU,)),
            pltpu.SemaphoreType.DMA((SUBU,)),
            pltpu.SemaphoreType.DMA((2,)),
            pltpu.SemaphoreType.DMA((4,)),
            pltpu.SemaphoreType.DMA((2,)),
        ],
        compiler_params=pltpu.CompilerParams(collective_id=0),
    )(x, W)
